# Initial kernel scaffold; baseline (speedup 1.0000x reference)
import jax
import jax.numpy as jnp
from jax.experimental import pallas as pl


def kernel(action, table):
    raise NotImplementedError("write your pallas kernel here")


# speedup vs baseline: 5.3144x; 5.3144x over previous
import jax
import jax.numpy as jnp
from jax.experimental import pallas as pl
from jax.experimental.pallas import tpu as pltpu
from jax.experimental.pallas import tpu_sc as plsc

DIM = 32
WINDOW = 128


def kernel(action, table):
    batch, seq = action.shape
    num_idx = batch * seq
    assert num_idx % WINDOW == 0
    grid = num_idx // WINDOW

    indices = action.reshape(1, num_idx)
    mesh = plsc.VectorSubcoreMesh(core_axis_name="c", subcore_axis_name="s")

    @jax.jit
    def gather(table_arr, idx_arr):
        @pl.kernel(
            out_type=jax.ShapeDtypeStruct((num_idx, DIM), table_arr.dtype),
            mesh=mesh,
            compiler_params=pltpu.CompilerParams(use_tc_tiling_on_sc=False),
        )
        def sc_gather(table_hbm, idx_hbm, out_hbm):
            def body(idx_vmem, out_vmem):
                pltpu.sync_copy(table_hbm.at[idx_vmem.at[0]], out_vmem)

            pltpu.emit_pipeline(
                body,
                grid=(grid,),
                in_specs=[pl.BlockSpec((1, WINDOW), index_map=lambda i: (0, i))],
                out_specs=[pl.BlockSpec((WINDOW, DIM), index_map=lambda i: (i, 0))],
                core_axis_name=("c", "s"),
                dimension_semantics=(pltpu.PARALLEL,),
            )(idx_hbm, out_hbm)

        return sc_gather(table_arr, idx_arr)

    out = gather(table, indices)
    return out.reshape(batch, seq, DIM)
